# Initial kernel scaffold; baseline (speedup 1.0000x reference)
import jax
import jax.numpy as jnp
from jax.experimental import pallas as pl


def kernel(x, adj, W1, b1, W2, b2, gpW1, gpb1, gpW2, gpb2):
    raise NotImplementedError("write your pallas kernel here")


# speedup vs baseline: 1.9922x; 1.9922x over previous
import jax
import jax.numpy as jnp
from jax.experimental import pallas as pl
from jax.experimental.pallas import tpu as pltpu

_TM = 400


def _gcn_body(x_ref, adj_ref, w1_ref, b1_ref, w2_ref, b2_ref,
              gw1_ref, gb1_ref, gw2_ref, gb2_ref,
              out_ref, sm_ref, s1_ref, h_ref, csum_ref):
    i = pl.program_id(0)
    n_steps = pl.num_programs(0)
    n_nodes = h_ref.shape[0]

    @pl.when(i == 0)
    def _init():
        s1_ref[...] = jnp.dot(x_ref[...], w1_ref[...],
                              preferred_element_type=jnp.float32)
        csum_ref[...] = jnp.zeros_like(csum_ref)

    a = adj_ref[...]
    h = jnp.dot(a, s1_ref[...], preferred_element_type=jnp.float32)
    h = jnp.maximum(h + b1_ref[...], 0.0)
    h_ref[pl.ds(i * _TM, _TM), :] = h
    csum_ref[...] += jnp.sum(a, axis=0, keepdims=True)

    @pl.when(i == n_steps - 1)
    def _tail():
        c = csum_ref[...] * (1.0 / n_nodes)
        v = jnp.dot(c, h_ref[...], preferred_element_type=jnp.float32)
        pooled = jnp.dot(v, w2_ref[...],
                         preferred_element_type=jnp.float32) + b2_ref[...]
        z = jnp.maximum(jnp.dot(pooled, gw1_ref[...],
                                preferred_element_type=jnp.float32)
                        + gb1_ref[...], 0.0)
        o = jnp.dot(z, gw2_ref[...],
                    preferred_element_type=jnp.float32) + gb2_ref[...]
        out_ref[...] = o
        m = jnp.max(o, axis=-1, keepdims=True)
        e = jnp.exp(o - m)
        sm_ref[...] = e / jnp.sum(e, axis=-1, keepdims=True)


def kernel(x, adj, W1, b1, W2, b2, gpW1, gpb1, gpW2, gpb2):
    n, d_feat = x.shape
    d_hid = W1.shape[1]
    n_class = W2.shape[1]
    gp_hid = gpW1.shape[1]
    n_steps = n // _TM

    out, out_sm = pl.pallas_call(
        _gcn_body,
        grid=(n_steps,),
        in_specs=[
            pl.BlockSpec((n, d_feat), lambda i: (0, 0)),
            pl.BlockSpec((_TM, n), lambda i: (i, 0)),
            pl.BlockSpec((d_feat, d_hid), lambda i: (0, 0)),
            pl.BlockSpec((1, d_hid), lambda i: (0, 0)),
            pl.BlockSpec((d_hid, n_class), lambda i: (0, 0)),
            pl.BlockSpec((1, n_class), lambda i: (0, 0)),
            pl.BlockSpec((n_class, gp_hid), lambda i: (0, 0)),
            pl.BlockSpec((1, gp_hid), lambda i: (0, 0)),
            pl.BlockSpec((gp_hid, n_class), lambda i: (0, 0)),
            pl.BlockSpec((1, n_class), lambda i: (0, 0)),
        ],
        out_specs=[
            pl.BlockSpec((1, n_class), lambda i: (0, 0)),
            pl.BlockSpec((1, n_class), lambda i: (0, 0)),
        ],
        out_shape=[
            jax.ShapeDtypeStruct((1, n_class), jnp.float32),
            jax.ShapeDtypeStruct((1, n_class), jnp.float32),
        ],
        scratch_shapes=[
            pltpu.VMEM((n, d_hid), jnp.float32),
            pltpu.VMEM((n, d_hid), jnp.float32),
            pltpu.VMEM((1, n), jnp.float32),
        ],
    )(x, adj, W1, b1.reshape(1, -1), W2, b2.reshape(1, -1),
      gpW1, gpb1.reshape(1, -1), gpW2, gpb2.reshape(1, -1))
    return out.reshape(-1), out_sm.reshape(-1)
